# baseline (speedup 1.0000x reference)
import functools

import jax
import jax.numpy as jnp
from jax import lax
from jax.experimental import pallas as pl
from jax.experimental.pallas import tpu as pltpu
from jax.experimental.pallas import tpu_sc as plsc

N = 10000
E = 320000
IN_DIM = 128
H1 = 64
H2 = 32

NC = 2
NS = 16
NW = NC * NS
EPW = E // NW
CHUNK = 125
NCHUNK = EPW // CHUNK
RPS = N // NS

DEC_TM = 400

_MESH = plsc.VectorSubcoreMesh(core_axis_name="c", subcore_axis_name="s")
_SC_PARAMS = pltpu.CompilerParams(use_tc_tiling_on_sc=False, needs_layout_passes=False)


@functools.partial(
    pl.kernel,
    out_type=jax.ShapeDtypeStruct((NC, N), jnp.float32),
    mesh=_MESH,
    compiler_params=_SC_PARAMS,
    scratch_types=[
        pltpu.VMEM((NCHUNK, CHUNK), jnp.int32),
        pltpu.VMEM((NCHUNK, CHUNK), jnp.float32),
        pltpu.VMEM_SHARED((N,), jnp.float32),
    ],
)
def _deg_sc(dstr, ewr, zcol, out, dstv, ewv, dacc):
    c = lax.axis_index("c")
    s = lax.axis_index("s")
    w = s * NC + c

    @pl.when(s == 0)
    def _init():
        pltpu.sync_copy(zcol, dacc)

    pltpu.sync_copy(dstr.at[w], dstv)
    pltpu.sync_copy(ewr.at[w], ewv)
    plsc.subcore_barrier()

    def chunk_body(j, carry):
        pltpu.sync_copy(ewv.at[j], dacc.at[dstv.at[j]], add=True)
        return carry

    lax.fori_loop(0, NCHUNK, chunk_body, 0)
    plsc.subcore_barrier()

    @pl.when(s == 0)
    def _flush():
        pltpu.sync_copy(dacc, out.at[c])


GB = CHUNK * H1 * 4
NPAIR = NCHUNK // 4


@functools.partial(
    pl.kernel,
    out_type=jax.ShapeDtypeStruct((NC, N, H1), jnp.float32),
    mesh=_MESH,
    compiler_params=_SC_PARAMS,
    scratch_types=[
        pltpu.VMEM((NCHUNK, CHUNK), jnp.int32),
        pltpu.VMEM((NCHUNK, CHUNK), jnp.int32),
        pltpu.VMEM((NCHUNK, CHUNK), jnp.float32),
        pltpu.VMEM((CHUNK, H1), jnp.float32),
        pltpu.VMEM((CHUNK, H1), jnp.float32),
        pltpu.VMEM((CHUNK, H1), jnp.float32),
        pltpu.VMEM((CHUNK, H1), jnp.float32),
        pltpu.VMEM_SHARED((N, H1), jnp.float32),
        pltpu.SemaphoreType.DMA,
        pltpu.SemaphoreType.DMA,
        pltpu.SemaphoreType.DMA,
        pltpu.SemaphoreType.DMA,
        pltpu.SemaphoreType.DMA,
        pltpu.SemaphoreType.DMA,
        pltpu.SemaphoreType.DMA,
        pltpu.SemaphoreType.DMA,
    ],
)
def _spass_sc(g, srcr, dstr, ewr, zrows, out, srcv, dstv, ewv,
              b0, b1, b2, b3, acc,
              sg0, sg1, sg2, sg3, ss0, ss1, ss2, ss3):
    c = lax.axis_index("c")
    s = lax.axis_index("s")
    w = s * NC + c
    bufs = (b0, b1, b2, b3)
    sgs = (sg0, sg1, sg2, sg3)
    sss = (ss0, ss1, ss2, ss3)

    pltpu.sync_copy(zrows, acc.at[pl.ds(s * RPS, RPS)])
    pltpu.sync_copy(srcr.at[w], srcv)
    pltpu.sync_copy(dstr.at[w], dstv)
    pltpu.sync_copy(ewr.at[w], ewv)
    plsc.subcore_barrier()

    def scale(buf, j):
        row = ewv.at[j]

        @plsc.parallel_loop(0, CHUNK, step=1, unroll=25)
        def _edge(e):
            wsp = plsc.load_gather(row, [jnp.full((16,), e, jnp.int32)])
            for q in range(H1 // 16):
                sl = pl.ds(q * 16, 16)
                buf[e, sl] = buf[e, sl] * wsp

    pltpu.async_copy(g.at[srcv.at[0]], b0, sg0)
    pltpu.async_copy(g.at[srcv.at[1]], b1, sg1)

    def drain(sem, buf):
        pltpu.make_async_copy(g.at[pl.ds(0, CHUNK)], buf, sem).wait()

    def ring_body(j, carry):
        t0 = 4 * j
        for u in range(4):
            t = t0 + u
            buf, sg, ss = bufs[u], sgs[u], sss[u]
            drain(sg, buf)
            scale(buf, t)
            pltpu.async_copy(buf, acc.at[dstv.at[t]], ss, add=True)
            v = (u + 2) % 4
            if u < 2:
                @pl.when(j > 0)
                def _wait_sc():
                    drain(sss[v], bufs[v])
                pltpu.async_copy(g.at[srcv.at[t + 2]], bufs[v], sgs[v])
            else:
                @pl.when(j < NPAIR - 1)
                def _rearm():
                    drain(sss[v], bufs[v])
                    pltpu.async_copy(g.at[srcv.at[t + 2]], bufs[v], sgs[v])
        return carry

    lax.fori_loop(0, NPAIR, ring_body, 0)
    for u in range(4):
        drain(sss[u], bufs[u])
    plsc.subcore_barrier()
    pltpu.sync_copy(acc.at[pl.ds(s * RPS, RPS)], out.at[c, pl.ds(s * RPS, RPS)])


def _prep_body(d0_ref, d1_ref, x_ref, w0_ref, dinv_ref, g0_ref):
    deg = d0_ref[...] + d1_ref[...] + 1.0
    dinv = jax.lax.rsqrt(deg)
    dinv_ref[...] = dinv
    h0 = jnp.dot(x_ref[...], w0_ref[...], preferred_element_type=jnp.float32)
    g0_ref[...] = h0 * dinv


def _prep(d0, d1, x, W0):
    TM = N
    grid = (N // TM,)
    return pl.pallas_call(
        _prep_body,
        grid=grid,
        in_specs=[
            pl.BlockSpec((TM, 1), lambda i: (i, 0)),
            pl.BlockSpec((TM, 1), lambda i: (i, 0)),
            pl.BlockSpec((TM, IN_DIM), lambda i: (i, 0)),
            pl.BlockSpec((IN_DIM, H1), lambda i: (0, 0)),
        ],
        out_specs=[
            pl.BlockSpec((TM, 1), lambda i: (i, 0)),
            pl.BlockSpec((TM, H1), lambda i: (i, 0)),
        ],
        out_shape=[
            jax.ShapeDtypeStruct((N, 1), jnp.float32),
            jax.ShapeDtypeStruct((N, H1), jnp.float32),
        ],
    )(d0, d1, x, W0)


def _mid_body(sp_ref, g0_ref, dinv_ref, b0_ref, wc_ref, g1_ref):
    dinv = dinv_ref[...]
    s0 = sp_ref[0] + sp_ref[1]
    a0 = dinv * (s0 + g0_ref[...]) + b0_ref[...].reshape(1, -1)
    h = jax.nn.relu(a0)
    h1 = jnp.dot(h, wc_ref[...], preferred_element_type=jnp.float32)
    g1_ref[...] = h1 * dinv


def _mid(sp, g0, dinv, b0, Wc):
    TM = N
    grid = (N // TM,)
    return pl.pallas_call(
        _mid_body,
        grid=grid,
        in_specs=[
            pl.BlockSpec((NC, TM, H1), lambda i: (0, i, 0)),
            pl.BlockSpec((TM, H1), lambda i: (i, 0)),
            pl.BlockSpec((TM, 1), lambda i: (i, 0)),
            pl.BlockSpec((H1,), lambda i: (0,)),
            pl.BlockSpec((H1, 2 * H2), lambda i: (0, 0)),
        ],
        out_specs=pl.BlockSpec((TM, 2 * H2), lambda i: (i, 0)),
        out_shape=jax.ShapeDtypeStruct((N, 2 * H2), jnp.float32),
    )(sp, g0, dinv, b0, Wc)


def _zstage_body(sp_ref, g1_ref, dinv_ref, b1_ref, b2_ref, noise_ref, z_ref):
    dinv = dinv_ref[...]
    s1 = sp_ref[0] + sp_ref[1]
    a1 = dinv * (s1 + g1_ref[...])
    mean = a1[:, :H2] + b1_ref[...].reshape(1, -1)
    log_std = a1[:, H2:] + b2_ref[...].reshape(1, -1)
    z_ref[...] = mean + noise_ref[...] * jnp.exp(log_std)


def _zstage(sp, g1, dinv, b1, b2, noise):
    TM = N
    grid = (N // TM,)
    return pl.pallas_call(
        _zstage_body,
        grid=grid,
        in_specs=[
            pl.BlockSpec((NC, TM, 2 * H2), lambda i: (0, i, 0)),
            pl.BlockSpec((TM, 2 * H2), lambda i: (i, 0)),
            pl.BlockSpec((TM, 1), lambda i: (i, 0)),
            pl.BlockSpec((H2,), lambda i: (0,)),
            pl.BlockSpec((H2,), lambda i: (0,)),
            pl.BlockSpec((TM, H2), lambda i: (i, 0)),
        ],
        out_specs=pl.BlockSpec((TM, H2), lambda i: (i, 0)),
        out_shape=jax.ShapeDtypeStruct((N, H2), jnp.float32),
    )(sp, g1, dinv, b1, b2, noise)


def _decoder_body(z_row_ref, z_all_ref, out_ref):
    zi = z_row_ref[...]
    zj = z_all_ref[...]
    acc = jax.lax.dot_general(zi, zj, (((1,), (1,)), ((), ())),
                              preferred_element_type=jnp.float32)
    out_ref[...] = 0.5 * jnp.tanh(acc * 0.5) + 0.5


def _decoder(z):
    grid = (N // DEC_TM,)
    return pl.pallas_call(
        _decoder_body,
        grid=grid,
        in_specs=[
            pl.BlockSpec((DEC_TM, H2), lambda i: (i, 0)),
            pl.BlockSpec((N, H2), lambda i: (0, 0)),
        ],
        out_specs=pl.BlockSpec((DEC_TM, N), lambda i: (i, 0)),
        out_shape=jax.ShapeDtypeStruct((N, N), jnp.float32),
    )(z, z)


@jax.jit
def kernel(x, edge_index, edge_attr, W0, b0, W1, b1, W2, b2, noise):
    srcr = edge_index[0].reshape(NW, NCHUNK, CHUNK)
    dstr = edge_index[1].reshape(NW, NCHUNK, CHUNK)
    ewr = edge_attr.reshape(NW, NCHUNK, CHUNK)
    zcol = jnp.zeros((N,), jnp.float32)
    zrows = jnp.zeros((RPS, H1), jnp.float32)

    degp = _deg_sc(dstr, ewr, zcol)
    dinv, g0 = _prep(degp[0].reshape(N, 1), degp[1].reshape(N, 1), x, W0)

    sp0 = _spass_sc(g0, srcr, dstr, ewr, zrows)

    Wc = jnp.concatenate([W1, W2], axis=1)
    g1 = _mid(sp0, g0, dinv, b0, Wc)

    sp1 = _spass_sc(g1, srcr, dstr, ewr, zrows)

    z = _zstage(sp1, g1, dinv, b1, b2, noise)
    return _decoder(z)


# speedup vs baseline: 1.0150x; 1.0150x over previous
import functools

import jax
import jax.numpy as jnp
from jax import lax
from jax.experimental import pallas as pl
from jax.experimental.pallas import tpu as pltpu
from jax.experimental.pallas import tpu_sc as plsc

N = 10000
E = 320000
IN_DIM = 128
H1 = 64
H2 = 32

NC = 2
NS = 16
NW = NC * NS
EPW = E // NW
CHUNK = 125
NCHUNK = EPW // CHUNK
RPS = N // NS

DEC_TM = 200

_MESH = plsc.VectorSubcoreMesh(core_axis_name="c", subcore_axis_name="s")
_SC_PARAMS = pltpu.CompilerParams(use_tc_tiling_on_sc=False, needs_layout_passes=False)


@functools.partial(
    pl.kernel,
    out_type=jax.ShapeDtypeStruct((NC, N), jnp.float32),
    mesh=_MESH,
    compiler_params=_SC_PARAMS,
    scratch_types=[
        pltpu.VMEM((NCHUNK, CHUNK), jnp.int32),
        pltpu.VMEM((NCHUNK, CHUNK), jnp.float32),
        pltpu.VMEM_SHARED((N,), jnp.float32),
    ],
)
def _deg_sc(dstr, ewr, zcol, out, dstv, ewv, dacc):
    c = lax.axis_index("c")
    s = lax.axis_index("s")
    w = s * NC + c

    @pl.when(s == 0)
    def _init():
        pltpu.sync_copy(zcol, dacc)

    pltpu.sync_copy(dstr.at[w], dstv)
    pltpu.sync_copy(ewr.at[w], ewv)
    plsc.subcore_barrier()

    def chunk_body(j, carry):
        pltpu.sync_copy(ewv.at[j], dacc.at[dstv.at[j]], add=True)
        return carry

    lax.fori_loop(0, NCHUNK, chunk_body, 0)
    plsc.subcore_barrier()

    @pl.when(s == 0)
    def _flush():
        pltpu.sync_copy(dacc, out.at[c])


GB = CHUNK * H1 * 4
NPAIR = NCHUNK // 4


@functools.partial(
    pl.kernel,
    out_type=jax.ShapeDtypeStruct((NC, N, H1), jnp.float32),
    mesh=_MESH,
    compiler_params=_SC_PARAMS,
    scratch_types=[
        pltpu.VMEM((NCHUNK, CHUNK), jnp.int32),
        pltpu.VMEM((NCHUNK, CHUNK), jnp.int32),
        pltpu.VMEM((NCHUNK, CHUNK), jnp.float32),
        pltpu.VMEM((CHUNK, H1), jnp.float32),
        pltpu.VMEM((CHUNK, H1), jnp.float32),
        pltpu.VMEM((CHUNK, H1), jnp.float32),
        pltpu.VMEM((CHUNK, H1), jnp.float32),
        pltpu.VMEM_SHARED((N, H1), jnp.float32),
        pltpu.SemaphoreType.DMA,
        pltpu.SemaphoreType.DMA,
        pltpu.SemaphoreType.DMA,
        pltpu.SemaphoreType.DMA,
        pltpu.SemaphoreType.DMA,
        pltpu.SemaphoreType.DMA,
        pltpu.SemaphoreType.DMA,
        pltpu.SemaphoreType.DMA,
    ],
)
def _spass_sc(g, srcr, dstr, ewr, zrows, out, srcv, dstv, ewv,
              b0, b1, b2, b3, acc,
              sg0, sg1, sg2, sg3, ss0, ss1, ss2, ss3):
    c = lax.axis_index("c")
    s = lax.axis_index("s")
    w = s * NC + c
    bufs = (b0, b1, b2, b3)
    sgs = (sg0, sg1, sg2, sg3)
    sss = (ss0, ss1, ss2, ss3)

    pltpu.sync_copy(zrows, acc.at[pl.ds(s * RPS, RPS)])
    pltpu.sync_copy(srcr.at[w], srcv)
    pltpu.sync_copy(dstr.at[w], dstv)
    pltpu.sync_copy(ewr.at[w], ewv)
    plsc.subcore_barrier()

    def scale(buf, j):
        row = ewv.at[j]

        @plsc.parallel_loop(0, CHUNK, step=1, unroll=5)
        def _edge(e):
            wsp = plsc.load_gather(row, [jnp.full((16,), e, jnp.int32)])
            for q in range(H1 // 16):
                sl = pl.ds(q * 16, 16)
                buf[e, sl] = buf[e, sl] * wsp

    pltpu.async_copy(g.at[srcv.at[0]], b0, sg0)
    pltpu.async_copy(g.at[srcv.at[1]], b1, sg1)

    def drain(sem, buf):
        pltpu.make_async_copy(g.at[pl.ds(0, CHUNK)], buf, sem).wait()

    def ring_body(j, carry):
        t0 = 4 * j
        for u in range(4):
            t = t0 + u
            buf, sg, ss = bufs[u], sgs[u], sss[u]
            drain(sg, buf)
            scale(buf, t)
            pltpu.async_copy(buf, acc.at[dstv.at[t]], ss, add=True)
            v = (u + 2) % 4
            if u < 2:
                @pl.when(j > 0)
                def _wait_sc():
                    drain(sss[v], bufs[v])
                pltpu.async_copy(g.at[srcv.at[t + 2]], bufs[v], sgs[v])
            else:
                @pl.when(j < NPAIR - 1)
                def _rearm():
                    drain(sss[v], bufs[v])
                    pltpu.async_copy(g.at[srcv.at[t + 2]], bufs[v], sgs[v])
        return carry

    lax.fori_loop(0, NPAIR, ring_body, 0)
    for u in range(4):
        drain(sss[u], bufs[u])
    plsc.subcore_barrier()
    pltpu.sync_copy(acc.at[pl.ds(s * RPS, RPS)], out.at[c, pl.ds(s * RPS, RPS)])


def _prep_body(d0_ref, d1_ref, x_ref, w0_ref, dinv_ref, g0_ref):
    deg = d0_ref[...] + d1_ref[...] + 1.0
    dinv = jax.lax.rsqrt(deg)
    dinv_ref[...] = dinv
    h0 = jnp.dot(x_ref[...], w0_ref[...], preferred_element_type=jnp.float32)
    g0_ref[...] = h0 * dinv


def _prep(d0, d1, x, W0):
    TM = N
    grid = (N // TM,)
    return pl.pallas_call(
        _prep_body,
        grid=grid,
        in_specs=[
            pl.BlockSpec((TM, 1), lambda i: (i, 0)),
            pl.BlockSpec((TM, 1), lambda i: (i, 0)),
            pl.BlockSpec((TM, IN_DIM), lambda i: (i, 0)),
            pl.BlockSpec((IN_DIM, H1), lambda i: (0, 0)),
        ],
        out_specs=[
            pl.BlockSpec((TM, 1), lambda i: (i, 0)),
            pl.BlockSpec((TM, H1), lambda i: (i, 0)),
        ],
        out_shape=[
            jax.ShapeDtypeStruct((N, 1), jnp.float32),
            jax.ShapeDtypeStruct((N, H1), jnp.float32),
        ],
    )(d0, d1, x, W0)


def _mid_body(sp_ref, g0_ref, dinv_ref, b0_ref, wc_ref, g1_ref):
    dinv = dinv_ref[...]
    s0 = sp_ref[0] + sp_ref[1]
    a0 = dinv * (s0 + g0_ref[...]) + b0_ref[...].reshape(1, -1)
    h = jax.nn.relu(a0)
    h1 = jnp.dot(h, wc_ref[...], preferred_element_type=jnp.float32)
    g1_ref[...] = h1 * dinv


def _mid(sp, g0, dinv, b0, Wc):
    TM = N
    grid = (N // TM,)
    return pl.pallas_call(
        _mid_body,
        grid=grid,
        in_specs=[
            pl.BlockSpec((NC, TM, H1), lambda i: (0, i, 0)),
            pl.BlockSpec((TM, H1), lambda i: (i, 0)),
            pl.BlockSpec((TM, 1), lambda i: (i, 0)),
            pl.BlockSpec((H1,), lambda i: (0,)),
            pl.BlockSpec((H1, 2 * H2), lambda i: (0, 0)),
        ],
        out_specs=pl.BlockSpec((TM, 2 * H2), lambda i: (i, 0)),
        out_shape=jax.ShapeDtypeStruct((N, 2 * H2), jnp.float32),
    )(sp, g0, dinv, b0, Wc)


def _zstage_body(sp_ref, g1_ref, dinv_ref, b1_ref, b2_ref, noise_ref, z_ref):
    dinv = dinv_ref[...]
    s1 = sp_ref[0] + sp_ref[1]
    a1 = dinv * (s1 + g1_ref[...])
    mean = a1[:, :H2] + b1_ref[...].reshape(1, -1)
    log_std = a1[:, H2:] + b2_ref[...].reshape(1, -1)
    z_ref[...] = mean + noise_ref[...] * jnp.exp(log_std)


def _zstage(sp, g1, dinv, b1, b2, noise):
    TM = N
    grid = (N // TM,)
    return pl.pallas_call(
        _zstage_body,
        grid=grid,
        in_specs=[
            pl.BlockSpec((NC, TM, 2 * H2), lambda i: (0, i, 0)),
            pl.BlockSpec((TM, 2 * H2), lambda i: (i, 0)),
            pl.BlockSpec((TM, 1), lambda i: (i, 0)),
            pl.BlockSpec((H2,), lambda i: (0,)),
            pl.BlockSpec((H2,), lambda i: (0,)),
            pl.BlockSpec((TM, H2), lambda i: (i, 0)),
        ],
        out_specs=pl.BlockSpec((TM, H2), lambda i: (i, 0)),
        out_shape=jax.ShapeDtypeStruct((N, H2), jnp.float32),
    )(sp, g1, dinv, b1, b2, noise)


def _decoder_body(sp_ref, g1_ref, dinv_ref, b1_ref, b2_ref, noise_ref,
                  out_ref, z_ref):
    i = pl.program_id(0)

    @pl.when(i == 0)
    def _compute_z():
        dinv = dinv_ref[...]
        a1 = dinv * (sp_ref[0] + sp_ref[1] + g1_ref[...])
        mean = a1[:, :H2] + b1_ref[...].reshape(1, -1)
        log_std = a1[:, H2:] + b2_ref[...].reshape(1, -1)
        z_ref[...] = mean + noise_ref[...] * jnp.exp(log_std)

    zi = z_ref[pl.ds(i * DEC_TM, DEC_TM), :]
    acc = jax.lax.dot_general(zi, z_ref[...], (((1,), (1,)), ((), ())),
                              preferred_element_type=jnp.float32)
    out_ref[...] = 0.5 * jnp.tanh(acc * 0.5) + 0.5


def _decoder(sp, g1, dinv, b1, b2, noise):
    grid = (N // DEC_TM,)
    return pl.pallas_call(
        _decoder_body,
        grid=grid,
        in_specs=[
            pl.BlockSpec((NC, N, 2 * H2), lambda i: (0, 0, 0)),
            pl.BlockSpec((N, 2 * H2), lambda i: (0, 0)),
            pl.BlockSpec((N, 1), lambda i: (0, 0)),
            pl.BlockSpec((H2,), lambda i: (0,)),
            pl.BlockSpec((H2,), lambda i: (0,)),
            pl.BlockSpec((N, H2), lambda i: (0, 0)),
        ],
        out_specs=pl.BlockSpec((DEC_TM, N), lambda i: (i, 0)),
        out_shape=jax.ShapeDtypeStruct((N, N), jnp.float32),
        scratch_shapes=[pltpu.VMEM((N, H2), jnp.float32)],
    )(sp, g1, dinv, b1, b2, noise)


@jax.jit
def kernel(x, edge_index, edge_attr, W0, b0, W1, b1, W2, b2, noise):
    srcr = edge_index[0].reshape(NW, NCHUNK, CHUNK)
    dstr = edge_index[1].reshape(NW, NCHUNK, CHUNK)
    ewr = edge_attr.reshape(NW, NCHUNK, CHUNK)
    zcol = jnp.zeros((N,), jnp.float32)
    zrows = jnp.zeros((RPS, H1), jnp.float32)

    degp = _deg_sc(dstr, ewr, zcol)
    dinv, g0 = _prep(degp[0].reshape(N, 1), degp[1].reshape(N, 1), x, W0)

    sp0 = _spass_sc(g0, srcr, dstr, ewr, zrows)

    Wc = jnp.concatenate([W1, W2], axis=1)
    g1 = _mid(sp0, g0, dinv, b0, Wc)

    sp1 = _spass_sc(g1, srcr, dstr, ewr, zrows)

    return _decoder(sp1, g1, dinv, b1, b2, noise)


# speedup vs baseline: 1.0168x; 1.0018x over previous
import functools

import jax
import jax.numpy as jnp
from jax import lax
from jax.experimental import pallas as pl
from jax.experimental.pallas import tpu as pltpu
from jax.experimental.pallas import tpu_sc as plsc

N = 10000
E = 320000
IN_DIM = 128
H1 = 64
H2 = 32

NC = 2
NS = 16
NW = NC * NS
EPW = E // NW
CHUNK = 125
NCHUNK = EPW // CHUNK
RPS = N // NS

DEC_TM = 200

_MESH = plsc.VectorSubcoreMesh(core_axis_name="c", subcore_axis_name="s")
_SC_PARAMS = pltpu.CompilerParams(use_tc_tiling_on_sc=False, needs_layout_passes=False)


@functools.partial(
    pl.kernel,
    out_type=jax.ShapeDtypeStruct((NC, N), jnp.float32),
    mesh=_MESH,
    compiler_params=_SC_PARAMS,
    scratch_types=[
        pltpu.VMEM((NCHUNK, CHUNK), jnp.int32),
        pltpu.VMEM((NCHUNK, CHUNK), jnp.float32),
        pltpu.VMEM_SHARED((N,), jnp.float32),
    ],
)
def _deg_sc(dstr, ewr, zcol, out, dstv, ewv, dacc):
    c = lax.axis_index("c")
    s = lax.axis_index("s")
    w = s * NC + c

    @pl.when(s == 0)
    def _init():
        pltpu.sync_copy(zcol, dacc)

    pltpu.sync_copy(dstr.at[w], dstv)
    pltpu.sync_copy(ewr.at[w], ewv)
    plsc.subcore_barrier()

    def chunk_body(j, carry):
        pltpu.sync_copy(ewv.at[j], dacc.at[dstv.at[j]], add=True)
        return carry

    lax.fori_loop(0, NCHUNK, chunk_body, 0)
    plsc.subcore_barrier()

    @pl.when(s == 0)
    def _flush():
        pltpu.sync_copy(dacc, out.at[c])


GB = CHUNK * H1 * 4
NPAIR = NCHUNK // 4


@functools.partial(
    pl.kernel,
    out_type=jax.ShapeDtypeStruct((NC, N, H1), jnp.float32),
    mesh=_MESH,
    compiler_params=_SC_PARAMS,
    scratch_types=[
        pltpu.VMEM((NCHUNK, CHUNK), jnp.int32),
        pltpu.VMEM((NCHUNK, CHUNK), jnp.int32),
        pltpu.VMEM((NCHUNK, CHUNK), jnp.float32),
        pltpu.VMEM((CHUNK, H1), jnp.float32),
        pltpu.VMEM((CHUNK, H1), jnp.float32),
        pltpu.VMEM((CHUNK, H1), jnp.float32),
        pltpu.VMEM((CHUNK, H1), jnp.float32),
        pltpu.VMEM_SHARED((N, H1), jnp.float32),
        pltpu.SemaphoreType.DMA,
        pltpu.SemaphoreType.DMA,
        pltpu.SemaphoreType.DMA,
        pltpu.SemaphoreType.DMA,
        pltpu.SemaphoreType.DMA,
        pltpu.SemaphoreType.DMA,
        pltpu.SemaphoreType.DMA,
        pltpu.SemaphoreType.DMA,
    ],
)
def _spass_sc(g, srcr, dstr, ewr, zrows, out, srcv, dstv, ewv,
              b0, b1, b2, b3, acc,
              sg0, sg1, sg2, sg3, ss0, ss1, ss2, ss3):
    c = lax.axis_index("c")
    s = lax.axis_index("s")
    w = s * NC + c
    bufs = (b0, b1, b2, b3)
    sgs = (sg0, sg1, sg2, sg3)
    sss = (ss0, ss1, ss2, ss3)

    pltpu.sync_copy(zrows, acc.at[pl.ds(s * RPS, RPS)])
    pltpu.sync_copy(srcr.at[w], srcv)
    pltpu.sync_copy(dstr.at[w], dstv)
    pltpu.sync_copy(ewr.at[w], ewv)
    plsc.subcore_barrier()

    def scale(buf, j):
        row = ewv.at[j]

        @plsc.parallel_loop(0, CHUNK, step=1, unroll=10)
        def _edge(e):
            wsp = plsc.load_gather(row, [jnp.full((16,), e, jnp.int32)])
            for q in range(H1 // 16):
                sl = pl.ds(q * 16, 16)
                buf[e, sl] = buf[e, sl] * wsp

    pltpu.async_copy(g.at[srcv.at[0]], b0, sg0)
    pltpu.async_copy(g.at[srcv.at[1]], b1, sg1)

    def drain(sem, buf):
        pltpu.make_async_copy(g.at[pl.ds(0, CHUNK)], buf, sem).wait()

    def ring_body(j, carry):
        t0 = 4 * j
        for u in range(4):
            t = t0 + u
            buf, sg, ss = bufs[u], sgs[u], sss[u]
            drain(sg, buf)
            scale(buf, t)
            pltpu.async_copy(buf, acc.at[dstv.at[t]], ss, add=True)
            v = (u + 2) % 4
            if u < 2:
                @pl.when(j > 0)
                def _wait_sc():
                    drain(sss[v], bufs[v])
                pltpu.async_copy(g.at[srcv.at[t + 2]], bufs[v], sgs[v])
            else:
                @pl.when(j < NPAIR - 1)
                def _rearm():
                    drain(sss[v], bufs[v])
                    pltpu.async_copy(g.at[srcv.at[t + 2]], bufs[v], sgs[v])
        return carry

    lax.fori_loop(0, NPAIR, ring_body, 0)
    for u in range(4):
        drain(sss[u], bufs[u])
    plsc.subcore_barrier()
    pltpu.sync_copy(acc.at[pl.ds(s * RPS, RPS)], out.at[c, pl.ds(s * RPS, RPS)])


def _prep_body(d0_ref, d1_ref, x_ref, w0_ref, dinv_ref, g0_ref):
    deg = d0_ref[...] + d1_ref[...] + 1.0
    dinv = jax.lax.rsqrt(deg)
    dinv_ref[...] = dinv
    h0 = jnp.dot(x_ref[...], w0_ref[...], preferred_element_type=jnp.float32)
    g0_ref[...] = h0 * dinv


def _prep(d0, d1, x, W0):
    TM = N
    grid = (N // TM,)
    return pl.pallas_call(
        _prep_body,
        grid=grid,
        in_specs=[
            pl.BlockSpec((TM, 1), lambda i: (i, 0)),
            pl.BlockSpec((TM, 1), lambda i: (i, 0)),
            pl.BlockSpec((TM, IN_DIM), lambda i: (i, 0)),
            pl.BlockSpec((IN_DIM, H1), lambda i: (0, 0)),
        ],
        out_specs=[
            pl.BlockSpec((TM, 1), lambda i: (i, 0)),
            pl.BlockSpec((TM, H1), lambda i: (i, 0)),
        ],
        out_shape=[
            jax.ShapeDtypeStruct((N, 1), jnp.float32),
            jax.ShapeDtypeStruct((N, H1), jnp.float32),
        ],
    )(d0, d1, x, W0)


def _mid_body(sp_ref, g0_ref, dinv_ref, b0_ref, wc_ref, g1_ref):
    dinv = dinv_ref[...]
    s0 = sp_ref[0] + sp_ref[1]
    a0 = dinv * (s0 + g0_ref[...]) + b0_ref[...].reshape(1, -1)
    h = jax.nn.relu(a0)
    h1 = jnp.dot(h, wc_ref[...], preferred_element_type=jnp.float32)
    g1_ref[...] = h1 * dinv


def _mid(sp, g0, dinv, b0, Wc):
    TM = N
    grid = (N // TM,)
    return pl.pallas_call(
        _mid_body,
        grid=grid,
        in_specs=[
            pl.BlockSpec((NC, TM, H1), lambda i: (0, i, 0)),
            pl.BlockSpec((TM, H1), lambda i: (i, 0)),
            pl.BlockSpec((TM, 1), lambda i: (i, 0)),
            pl.BlockSpec((H1,), lambda i: (0,)),
            pl.BlockSpec((H1, 2 * H2), lambda i: (0, 0)),
        ],
        out_specs=pl.BlockSpec((TM, 2 * H2), lambda i: (i, 0)),
        out_shape=jax.ShapeDtypeStruct((N, 2 * H2), jnp.float32),
    )(sp, g0, dinv, b0, Wc)


def _zstage_body(sp_ref, g1_ref, dinv_ref, b1_ref, b2_ref, noise_ref, z_ref):
    dinv = dinv_ref[...]
    s1 = sp_ref[0] + sp_ref[1]
    a1 = dinv * (s1 + g1_ref[...])
    mean = a1[:, :H2] + b1_ref[...].reshape(1, -1)
    log_std = a1[:, H2:] + b2_ref[...].reshape(1, -1)
    z_ref[...] = mean + noise_ref[...] * jnp.exp(log_std)


def _zstage(sp, g1, dinv, b1, b2, noise):
    TM = N
    grid = (N // TM,)
    return pl.pallas_call(
        _zstage_body,
        grid=grid,
        in_specs=[
            pl.BlockSpec((NC, TM, 2 * H2), lambda i: (0, i, 0)),
            pl.BlockSpec((TM, 2 * H2), lambda i: (i, 0)),
            pl.BlockSpec((TM, 1), lambda i: (i, 0)),
            pl.BlockSpec((H2,), lambda i: (0,)),
            pl.BlockSpec((H2,), lambda i: (0,)),
            pl.BlockSpec((TM, H2), lambda i: (i, 0)),
        ],
        out_specs=pl.BlockSpec((TM, H2), lambda i: (i, 0)),
        out_shape=jax.ShapeDtypeStruct((N, H2), jnp.float32),
    )(sp, g1, dinv, b1, b2, noise)


def _decoder_body(sp_ref, g1_ref, dinv_ref, b1_ref, b2_ref, noise_ref,
                  out_ref, z_ref):
    i = pl.program_id(0)

    @pl.when(i == 0)
    def _compute_z():
        dinv = dinv_ref[...]
        a1 = dinv * (sp_ref[0] + sp_ref[1] + g1_ref[...])
        mean = a1[:, :H2] + b1_ref[...].reshape(1, -1)
        log_std = a1[:, H2:] + b2_ref[...].reshape(1, -1)
        z_ref[...] = mean + noise_ref[...] * jnp.exp(log_std)

    zi = z_ref[pl.ds(i * DEC_TM, DEC_TM), :]
    acc = jax.lax.dot_general(zi, z_ref[...], (((1,), (1,)), ((), ())),
                              preferred_element_type=jnp.float32)
    out_ref[...] = 0.5 * jnp.tanh(acc * 0.5) + 0.5


def _decoder(sp, g1, dinv, b1, b2, noise):
    grid = (N // DEC_TM,)
    return pl.pallas_call(
        _decoder_body,
        grid=grid,
        in_specs=[
            pl.BlockSpec((NC, N, 2 * H2), lambda i: (0, 0, 0)),
            pl.BlockSpec((N, 2 * H2), lambda i: (0, 0)),
            pl.BlockSpec((N, 1), lambda i: (0, 0)),
            pl.BlockSpec((H2,), lambda i: (0,)),
            pl.BlockSpec((H2,), lambda i: (0,)),
            pl.BlockSpec((N, H2), lambda i: (0, 0)),
        ],
        out_specs=pl.BlockSpec((DEC_TM, N), lambda i: (i, 0)),
        out_shape=jax.ShapeDtypeStruct((N, N), jnp.float32),
        scratch_shapes=[pltpu.VMEM((N, H2), jnp.float32)],
    )(sp, g1, dinv, b1, b2, noise)


@jax.jit
def kernel(x, edge_index, edge_attr, W0, b0, W1, b1, W2, b2, noise):
    srcr = edge_index[0].reshape(NW, NCHUNK, CHUNK)
    dstr = edge_index[1].reshape(NW, NCHUNK, CHUNK)
    ewr = edge_attr.reshape(NW, NCHUNK, CHUNK)
    zcol = jnp.zeros((N,), jnp.float32)
    zrows = jnp.zeros((RPS, H1), jnp.float32)

    degp = _deg_sc(dstr, ewr, zcol)
    dinv, g0 = _prep(degp[0].reshape(N, 1), degp[1].reshape(N, 1), x, W0)

    sp0 = _spass_sc(g0, srcr, dstr, ewr, zrows)

    Wc = jnp.concatenate([W1, W2], axis=1)
    g1 = _mid(sp0, g0, dinv, b0, Wc)

    sp1 = _spass_sc(g1, srcr, dstr, ewr, zrows)

    return _decoder(sp1, g1, dinv, b1, b2, noise)


# speedup vs baseline: 1.0527x; 1.0353x over previous
import functools

import jax
import jax.numpy as jnp
from jax import lax
from jax.experimental import pallas as pl
from jax.experimental.pallas import tpu as pltpu
from jax.experimental.pallas import tpu_sc as plsc

N = 10000
E = 320000
IN_DIM = 128
H1 = 64
H2 = 32

NC = 2
NS = 16
NW = NC * NS
EPW = E // NW
CHUNK = 125
NCHUNK = EPW // CHUNK
RPS = N // NS

DEC_TM = 200

_MESH = plsc.VectorSubcoreMesh(core_axis_name="c", subcore_axis_name="s")
_SC_PARAMS = pltpu.CompilerParams(use_tc_tiling_on_sc=False, needs_layout_passes=False)


@functools.partial(
    pl.kernel,
    out_type=jax.ShapeDtypeStruct((NC, N), jnp.float32),
    mesh=_MESH,
    compiler_params=_SC_PARAMS,
    scratch_types=[
        pltpu.VMEM((NCHUNK, CHUNK), jnp.int32),
        pltpu.VMEM((NCHUNK, CHUNK), jnp.float32),
        pltpu.VMEM_SHARED((N,), jnp.float32),
    ],
)
def _deg_sc(dstr, ewr, zcol, out, dstv, ewv, dacc):
    c = lax.axis_index("c")
    s = lax.axis_index("s")
    w = s * NC + c

    @pl.when(s == 0)
    def _init():
        pltpu.sync_copy(zcol, dacc)

    pltpu.sync_copy(dstr.at[w], dstv)
    pltpu.sync_copy(ewr.at[w], ewv)
    plsc.subcore_barrier()

    def chunk_body(j, carry):
        pltpu.sync_copy(ewv.at[j], dacc.at[dstv.at[j]], add=True)
        return carry

    lax.fori_loop(0, NCHUNK, chunk_body, 0)
    plsc.subcore_barrier()

    @pl.when(s == 0)
    def _flush():
        pltpu.sync_copy(dacc, out.at[c])


GB = CHUNK * H1 * 4
NRING = 5
NITER = NCHUNK // NRING
LEAD = 3


@functools.partial(
    pl.kernel,
    out_type=jax.ShapeDtypeStruct((NC, N, H1), jnp.float32),
    mesh=_MESH,
    compiler_params=_SC_PARAMS,
    scratch_types=(
        [
            pltpu.VMEM((NCHUNK, CHUNK), jnp.int32),
            pltpu.VMEM((NCHUNK, CHUNK), jnp.int32),
            pltpu.VMEM((NCHUNK, CHUNK), jnp.float32),
        ]
        + [pltpu.VMEM((CHUNK, H1), jnp.float32)] * NRING
        + [pltpu.VMEM_SHARED((N, H1), jnp.float32)]
        + [pltpu.SemaphoreType.DMA] * (2 * NRING)
    ),
)
def _spass_sc(g, srcr, dstr, ewr, zrows, out, srcv, dstv, ewv, *rest):
    bufs = rest[:NRING]
    acc = rest[NRING]
    sgs = rest[NRING + 1:2 * NRING + 1]
    sss = rest[2 * NRING + 1:]
    c = lax.axis_index("c")
    s = lax.axis_index("s")
    w = s * NC + c

    pltpu.sync_copy(zrows, acc.at[pl.ds(s * RPS, RPS)])
    pltpu.sync_copy(srcr.at[w], srcv)
    pltpu.sync_copy(dstr.at[w], dstv)
    pltpu.sync_copy(ewr.at[w], ewv)
    plsc.subcore_barrier()

    def scale(buf, j):
        row = ewv.at[j]

        @plsc.parallel_loop(0, CHUNK, step=1, unroll=10)
        def _edge(e):
            wsp = plsc.load_gather(row, [jnp.full((16,), e, jnp.int32)])
            for q in range(H1 // 16):
                sl = pl.ds(q * 16, 16)
                buf[e, sl] = buf[e, sl] * wsp

    for u in range(LEAD):
        pltpu.async_copy(g.at[srcv.at[u]], bufs[u], sgs[u])

    def drain(sem, buf):
        pltpu.make_async_copy(g.at[pl.ds(0, CHUNK)], buf, sem).wait()

    def ring_body(j, carry):
        t0 = NRING * j
        for u in range(NRING):
            t = t0 + u
            buf, sg, ss = bufs[u], sgs[u], sss[u]
            drain(sg, buf)
            scale(buf, t)
            pltpu.async_copy(buf, acc.at[dstv.at[t]], ss, add=True)
            v = (u + LEAD) % NRING
            if u < NRING - LEAD:
                @pl.when(j > 0)
                def _wait_sc():
                    drain(sss[v], bufs[v])
                pltpu.async_copy(g.at[srcv.at[t + LEAD]], bufs[v], sgs[v])
            else:
                @pl.when(j < NITER - 1)
                def _rearm():
                    drain(sss[v], bufs[v])
                    pltpu.async_copy(g.at[srcv.at[t + LEAD]], bufs[v], sgs[v])
        return carry

    lax.fori_loop(0, NITER, ring_body, 0)
    for u in range(NRING):
        drain(sss[u], bufs[u])
    plsc.subcore_barrier()
    pltpu.sync_copy(acc.at[pl.ds(s * RPS, RPS)], out.at[c, pl.ds(s * RPS, RPS)])


def _prep_body(d0_ref, d1_ref, x_ref, w0_ref, dinv_ref, g0_ref):
    deg = d0_ref[...] + d1_ref[...] + 1.0
    dinv = jax.lax.rsqrt(deg)
    dinv_ref[...] = dinv
    h0 = jnp.dot(x_ref[...], w0_ref[...], preferred_element_type=jnp.float32)
    g0_ref[...] = h0 * dinv


def _prep(d0, d1, x, W0):
    TM = N
    grid = (N // TM,)
    return pl.pallas_call(
        _prep_body,
        grid=grid,
        in_specs=[
            pl.BlockSpec((TM, 1), lambda i: (i, 0)),
            pl.BlockSpec((TM, 1), lambda i: (i, 0)),
            pl.BlockSpec((TM, IN_DIM), lambda i: (i, 0)),
            pl.BlockSpec((IN_DIM, H1), lambda i: (0, 0)),
        ],
        out_specs=[
            pl.BlockSpec((TM, 1), lambda i: (i, 0)),
            pl.BlockSpec((TM, H1), lambda i: (i, 0)),
        ],
        out_shape=[
            jax.ShapeDtypeStruct((N, 1), jnp.float32),
            jax.ShapeDtypeStruct((N, H1), jnp.float32),
        ],
    )(d0, d1, x, W0)


def _mid_body(sp_ref, g0_ref, dinv_ref, b0_ref, wc_ref, g1_ref):
    dinv = dinv_ref[...]
    s0 = sp_ref[0] + sp_ref[1]
    a0 = dinv * (s0 + g0_ref[...]) + b0_ref[...].reshape(1, -1)
    h = jax.nn.relu(a0)
    h1 = jnp.dot(h, wc_ref[...], preferred_element_type=jnp.float32)
    g1_ref[...] = h1 * dinv


def _mid(sp, g0, dinv, b0, Wc):
    TM = N
    grid = (N // TM,)
    return pl.pallas_call(
        _mid_body,
        grid=grid,
        in_specs=[
            pl.BlockSpec((NC, TM, H1), lambda i: (0, i, 0)),
            pl.BlockSpec((TM, H1), lambda i: (i, 0)),
            pl.BlockSpec((TM, 1), lambda i: (i, 0)),
            pl.BlockSpec((H1,), lambda i: (0,)),
            pl.BlockSpec((H1, 2 * H2), lambda i: (0, 0)),
        ],
        out_specs=pl.BlockSpec((TM, 2 * H2), lambda i: (i, 0)),
        out_shape=jax.ShapeDtypeStruct((N, 2 * H2), jnp.float32),
    )(sp, g0, dinv, b0, Wc)


def _zstage_body(sp_ref, g1_ref, dinv_ref, b1_ref, b2_ref, noise_ref, z_ref):
    dinv = dinv_ref[...]
    s1 = sp_ref[0] + sp_ref[1]
    a1 = dinv * (s1 + g1_ref[...])
    mean = a1[:, :H2] + b1_ref[...].reshape(1, -1)
    log_std = a1[:, H2:] + b2_ref[...].reshape(1, -1)
    z_ref[...] = mean + noise_ref[...] * jnp.exp(log_std)


def _zstage(sp, g1, dinv, b1, b2, noise):
    TM = N
    grid = (N // TM,)
    return pl.pallas_call(
        _zstage_body,
        grid=grid,
        in_specs=[
            pl.BlockSpec((NC, TM, 2 * H2), lambda i: (0, i, 0)),
            pl.BlockSpec((TM, 2 * H2), lambda i: (i, 0)),
            pl.BlockSpec((TM, 1), lambda i: (i, 0)),
            pl.BlockSpec((H2,), lambda i: (0,)),
            pl.BlockSpec((H2,), lambda i: (0,)),
            pl.BlockSpec((TM, H2), lambda i: (i, 0)),
        ],
        out_specs=pl.BlockSpec((TM, H2), lambda i: (i, 0)),
        out_shape=jax.ShapeDtypeStruct((N, H2), jnp.float32),
    )(sp, g1, dinv, b1, b2, noise)


def _decoder_body(sp_ref, g1_ref, dinv_ref, b1_ref, b2_ref, noise_ref,
                  out_ref, z_ref):
    i = pl.program_id(0)

    @pl.when(i == 0)
    def _compute_z():
        dinv = dinv_ref[...]
        a1 = dinv * (sp_ref[0] + sp_ref[1] + g1_ref[...])
        mean = a1[:, :H2] + b1_ref[...].reshape(1, -1)
        log_std = a1[:, H2:] + b2_ref[...].reshape(1, -1)
        z_ref[...] = mean + noise_ref[...] * jnp.exp(log_std)

    zi = z_ref[pl.ds(i * DEC_TM, DEC_TM), :]
    acc = jax.lax.dot_general(zi, z_ref[...], (((1,), (1,)), ((), ())),
                              preferred_element_type=jnp.float32)
    out_ref[...] = 0.5 * jnp.tanh(acc * 0.5) + 0.5


def _decoder(sp, g1, dinv, b1, b2, noise):
    grid = (N // DEC_TM,)
    return pl.pallas_call(
        _decoder_body,
        grid=grid,
        in_specs=[
            pl.BlockSpec((NC, N, 2 * H2), lambda i: (0, 0, 0)),
            pl.BlockSpec((N, 2 * H2), lambda i: (0, 0)),
            pl.BlockSpec((N, 1), lambda i: (0, 0)),
            pl.BlockSpec((H2,), lambda i: (0,)),
            pl.BlockSpec((H2,), lambda i: (0,)),
            pl.BlockSpec((N, H2), lambda i: (0, 0)),
        ],
        out_specs=pl.BlockSpec((DEC_TM, N), lambda i: (i, 0)),
        out_shape=jax.ShapeDtypeStruct((N, N), jnp.float32),
        scratch_shapes=[pltpu.VMEM((N, H2), jnp.float32)],
    )(sp, g1, dinv, b1, b2, noise)


@jax.jit
def kernel(x, edge_index, edge_attr, W0, b0, W1, b1, W2, b2, noise):
    srcr = edge_index[0].reshape(NW, NCHUNK, CHUNK)
    dstr = edge_index[1].reshape(NW, NCHUNK, CHUNK)
    ewr = edge_attr.reshape(NW, NCHUNK, CHUNK)
    zcol = jnp.zeros((N,), jnp.float32)
    zrows = jnp.zeros((RPS, H1), jnp.float32)

    degp = _deg_sc(dstr, ewr, zcol)
    dinv, g0 = _prep(degp[0].reshape(N, 1), degp[1].reshape(N, 1), x, W0)

    sp0 = _spass_sc(g0, srcr, dstr, ewr, zrows)

    Wc = jnp.concatenate([W1, W2], axis=1)
    g1 = _mid(sp0, g0, dinv, b0, Wc)

    sp1 = _spass_sc(g1, srcr, dstr, ewr, zrows)

    return _decoder(sp1, g1, dinv, b1, b2, noise)


# speedup vs baseline: 1.0532x; 1.0005x over previous
import functools

import jax
import jax.numpy as jnp
from jax import lax
from jax.experimental import pallas as pl
from jax.experimental.pallas import tpu as pltpu
from jax.experimental.pallas import tpu_sc as plsc

N = 10000
E = 320000
IN_DIM = 128
H1 = 64
H2 = 32

NC = 2
NS = 16
NW = NC * NS
EPW = E // NW
CHUNK = 125
NCHUNK = EPW // CHUNK
RPS = N // NS

DEC_TM = 200

_MESH = plsc.VectorSubcoreMesh(core_axis_name="c", subcore_axis_name="s")
_SC_PARAMS = pltpu.CompilerParams(use_tc_tiling_on_sc=False, needs_layout_passes=False)


@functools.partial(
    pl.kernel,
    out_type=jax.ShapeDtypeStruct((NC, N), jnp.float32),
    mesh=_MESH,
    compiler_params=_SC_PARAMS,
    scratch_types=[
        pltpu.VMEM((NCHUNK, CHUNK), jnp.int32),
        pltpu.VMEM((NCHUNK, CHUNK), jnp.float32),
        pltpu.VMEM_SHARED((N,), jnp.float32),
    ],
)
def _deg_sc(dstr, ewr, zcol, out, dstv, ewv, dacc):
    c = lax.axis_index("c")
    s = lax.axis_index("s")
    w = s * NC + c

    @pl.when(s == 0)
    def _init():
        pltpu.sync_copy(zcol, dacc)

    pltpu.sync_copy(dstr.at[w], dstv)
    pltpu.sync_copy(ewr.at[w], ewv)
    plsc.subcore_barrier()

    def chunk_body(j, carry):
        pltpu.sync_copy(ewv.at[j], dacc.at[dstv.at[j]], add=True)
        return carry

    lax.fori_loop(0, NCHUNK, chunk_body, 0)
    plsc.subcore_barrier()

    @pl.when(s == 0)
    def _flush():
        pltpu.sync_copy(dacc, out.at[c])


GB = CHUNK * H1 * 4
NRING = 5
NITER = NCHUNK // NRING
LEAD = 3


@functools.partial(
    pl.kernel,
    out_type=jax.ShapeDtypeStruct((NC, N, H1), jnp.float32),
    mesh=_MESH,
    compiler_params=_SC_PARAMS,
    scratch_types=(
        [
            pltpu.VMEM((NCHUNK, CHUNK), jnp.int32),
            pltpu.VMEM((NCHUNK, CHUNK), jnp.int32),
            pltpu.VMEM((NCHUNK, CHUNK), jnp.float32),
        ]
        + [pltpu.VMEM((CHUNK, H1), jnp.float32)] * NRING
        + [pltpu.VMEM_SHARED((N, H1), jnp.float32)]
        + [pltpu.SemaphoreType.DMA] * (2 * NRING)
    ),
)
def _spass_sc(g, srcr, dstr, ewr, zrows, out, srcv, dstv, ewv, *rest):
    bufs = rest[:NRING]
    acc = rest[NRING]
    sgs = rest[NRING + 1:2 * NRING + 1]
    sss = rest[2 * NRING + 1:]
    c = lax.axis_index("c")
    s = lax.axis_index("s")
    w = s * NC + c

    pltpu.sync_copy(zrows, acc.at[pl.ds(s * RPS, RPS)])
    pltpu.sync_copy(srcr.at[w], srcv)
    pltpu.sync_copy(dstr.at[w], dstv)
    pltpu.sync_copy(ewr.at[w], ewv)
    plsc.subcore_barrier()

    def scale(buf, j):
        row = ewv.at[j]

        @plsc.parallel_loop(0, CHUNK, step=1, unroll=10)
        def _edge(e):
            wsp = plsc.load_gather(row, [jnp.full((16,), e, jnp.int32)])
            for q in range(H1 // 16):
                sl = pl.ds(q * 16, 16)
                buf[e, sl] = buf[e, sl] * wsp

    for u in range(LEAD):
        pltpu.async_copy(g.at[srcv.at[u]], bufs[u], sgs[u])

    def drain(sem, buf):
        pltpu.make_async_copy(g.at[pl.ds(0, CHUNK)], buf, sem).wait()

    def ring_body(j, carry):
        t0 = NRING * j
        for u in range(NRING):
            t = t0 + u
            buf, sg, ss = bufs[u], sgs[u], sss[u]
            drain(sg, buf)
            scale(buf, t)
            pltpu.async_copy(buf, acc.at[dstv.at[t]], ss, add=True)
            v = (u + LEAD) % NRING
            if u < NRING - LEAD:
                @pl.when(j > 0)
                def _wait_sc():
                    drain(sss[v], bufs[v])
                pltpu.async_copy(g.at[srcv.at[t + LEAD]], bufs[v], sgs[v])
            else:
                @pl.when(j < NITER - 1)
                def _rearm():
                    drain(sss[v], bufs[v])
                    pltpu.async_copy(g.at[srcv.at[t + LEAD]], bufs[v], sgs[v])
        return carry

    lax.fori_loop(0, NITER, ring_body, 0)
    for u in range(NRING):
        drain(sss[u], bufs[u])
    plsc.subcore_barrier()
    pltpu.sync_copy(acc.at[pl.ds(s * RPS, RPS)], out.at[c, pl.ds(s * RPS, RPS)])


def _xw0_body(x_ref, w0_ref, h0_ref):
    h0_ref[...] = jnp.dot(x_ref[...], w0_ref[...],
                          preferred_element_type=jnp.float32)


def _xw0(x, W0):
    return pl.pallas_call(
        _xw0_body,
        grid=(1,),
        in_specs=[
            pl.BlockSpec((N, IN_DIM), lambda i: (0, 0)),
            pl.BlockSpec((IN_DIM, H1), lambda i: (0, 0)),
        ],
        out_specs=pl.BlockSpec((N, H1), lambda i: (0, 0)),
        out_shape=jax.ShapeDtypeStruct((N, H1), jnp.float32),
    )(x, W0)


def _prep_body(d0_ref, d1_ref, h0_ref, dinv_ref, g0_ref):
    deg = d0_ref[...] + d1_ref[...] + 1.0
    dinv = jax.lax.rsqrt(deg)
    dinv_ref[...] = dinv
    g0_ref[...] = h0_ref[...] * dinv


def _prep(d0, d1, h0):
    return pl.pallas_call(
        _prep_body,
        grid=(1,),
        in_specs=[
            pl.BlockSpec((N, 1), lambda i: (0, 0)),
            pl.BlockSpec((N, 1), lambda i: (0, 0)),
            pl.BlockSpec((N, H1), lambda i: (0, 0)),
        ],
        out_specs=[
            pl.BlockSpec((N, 1), lambda i: (0, 0)),
            pl.BlockSpec((N, H1), lambda i: (0, 0)),
        ],
        out_shape=[
            jax.ShapeDtypeStruct((N, 1), jnp.float32),
            jax.ShapeDtypeStruct((N, H1), jnp.float32),
        ],
    )(d0, d1, h0)


def _mid_body(sp_ref, g0_ref, dinv_ref, b0_ref, wc_ref, g1_ref):
    dinv = dinv_ref[...]
    s0 = sp_ref[0] + sp_ref[1]
    a0 = dinv * (s0 + g0_ref[...]) + b0_ref[...].reshape(1, -1)
    h = jax.nn.relu(a0)
    h1 = jnp.dot(h, wc_ref[...], preferred_element_type=jnp.float32)
    g1_ref[...] = h1 * dinv


def _mid(sp, g0, dinv, b0, Wc):
    TM = N
    grid = (N // TM,)
    return pl.pallas_call(
        _mid_body,
        grid=grid,
        in_specs=[
            pl.BlockSpec((NC, TM, H1), lambda i: (0, i, 0)),
            pl.BlockSpec((TM, H1), lambda i: (i, 0)),
            pl.BlockSpec((TM, 1), lambda i: (i, 0)),
            pl.BlockSpec((H1,), lambda i: (0,)),
            pl.BlockSpec((H1, 2 * H2), lambda i: (0, 0)),
        ],
        out_specs=pl.BlockSpec((TM, 2 * H2), lambda i: (i, 0)),
        out_shape=jax.ShapeDtypeStruct((N, 2 * H2), jnp.float32),
    )(sp, g0, dinv, b0, Wc)


def _zstage_body(sp_ref, g1_ref, dinv_ref, b1_ref, b2_ref, noise_ref, z_ref):
    dinv = dinv_ref[...]
    s1 = sp_ref[0] + sp_ref[1]
    a1 = dinv * (s1 + g1_ref[...])
    mean = a1[:, :H2] + b1_ref[...].reshape(1, -1)
    log_std = a1[:, H2:] + b2_ref[...].reshape(1, -1)
    z_ref[...] = mean + noise_ref[...] * jnp.exp(log_std)


def _zstage(sp, g1, dinv, b1, b2, noise):
    TM = N
    grid = (N // TM,)
    return pl.pallas_call(
        _zstage_body,
        grid=grid,
        in_specs=[
            pl.BlockSpec((NC, TM, 2 * H2), lambda i: (0, i, 0)),
            pl.BlockSpec((TM, 2 * H2), lambda i: (i, 0)),
            pl.BlockSpec((TM, 1), lambda i: (i, 0)),
            pl.BlockSpec((H2,), lambda i: (0,)),
            pl.BlockSpec((H2,), lambda i: (0,)),
            pl.BlockSpec((TM, H2), lambda i: (i, 0)),
        ],
        out_specs=pl.BlockSpec((TM, H2), lambda i: (i, 0)),
        out_shape=jax.ShapeDtypeStruct((N, H2), jnp.float32),
    )(sp, g1, dinv, b1, b2, noise)


def _decoder_body(sp_ref, g1_ref, dinv_ref, b1_ref, b2_ref, noise_ref,
                  out_ref, z_ref):
    i = pl.program_id(0)

    @pl.when(i == 0)
    def _compute_z():
        dinv = dinv_ref[...]
        a1 = dinv * (sp_ref[0] + sp_ref[1] + g1_ref[...])
        mean = a1[:, :H2] + b1_ref[...].reshape(1, -1)
        log_std = a1[:, H2:] + b2_ref[...].reshape(1, -1)
        z_ref[...] = mean + noise_ref[...] * jnp.exp(log_std)

    zi = z_ref[pl.ds(i * DEC_TM, DEC_TM), :]
    acc = jax.lax.dot_general(zi, z_ref[...], (((1,), (1,)), ((), ())),
                              preferred_element_type=jnp.float32)
    out_ref[...] = 0.5 * jnp.tanh(acc * 0.5) + 0.5


def _decoder(sp, g1, dinv, b1, b2, noise):
    grid = (N // DEC_TM,)
    return pl.pallas_call(
        _decoder_body,
        grid=grid,
        in_specs=[
            pl.BlockSpec((NC, N, 2 * H2), lambda i: (0, 0, 0)),
            pl.BlockSpec((N, 2 * H2), lambda i: (0, 0)),
            pl.BlockSpec((N, 1), lambda i: (0, 0)),
            pl.BlockSpec((H2,), lambda i: (0,)),
            pl.BlockSpec((H2,), lambda i: (0,)),
            pl.BlockSpec((N, H2), lambda i: (0, 0)),
        ],
        out_specs=pl.BlockSpec((DEC_TM, N), lambda i: (i, 0)),
        out_shape=jax.ShapeDtypeStruct((N, N), jnp.float32),
        scratch_shapes=[pltpu.VMEM((N, H2), jnp.float32)],
    )(sp, g1, dinv, b1, b2, noise)


@jax.jit
def kernel(x, edge_index, edge_attr, W0, b0, W1, b1, W2, b2, noise):
    srcr = edge_index[0].reshape(NW, NCHUNK, CHUNK)
    dstr = edge_index[1].reshape(NW, NCHUNK, CHUNK)
    ewr = edge_attr.reshape(NW, NCHUNK, CHUNK)
    zcol = jnp.zeros((N,), jnp.float32)
    zrows = jnp.zeros((RPS, H1), jnp.float32)

    h0 = _xw0(x, W0)
    degp = _deg_sc(dstr, ewr, zcol)
    dinv, g0 = _prep(degp[0].reshape(N, 1), degp[1].reshape(N, 1), h0)

    sp0 = _spass_sc(g0, srcr, dstr, ewr, zrows)

    Wc = jnp.concatenate([W1, W2], axis=1)
    g1 = _mid(sp0, g0, dinv, b0, Wc)

    sp1 = _spass_sc(g1, srcr, dstr, ewr, zrows)

    return _decoder(sp1, g1, dinv, b1, b2, noise)
